# baseline (speedup 1.0000x reference)
import functools

import jax
import jax.numpy as jnp
from jax import lax
from jax.experimental import pallas as pl
from jax.experimental.pallas import tpu as pltpu
from jax.experimental.pallas import tpu_sc as plsc

_NUM_CORES = 2
_NUM_SUBCORES = 16
_NW = _NUM_CORES * _NUM_SUBCORES
_R = 16


def _sc_add(x_flat, pe_flat, b_sc, s, d):
    pe_rows_w = s // _NW
    n_chunks = pe_rows_w // _R
    T = n_chunks * b_sc
    blk = _R * d

    def body(x_hbm, pe_hbm, o_hbm, xb0, xb1, xb2, xb3, pb0, pb1,
             sx0, sx1, sx2, sx3, sp0, sp1, so0, so1, so2, so3):
        xbuf = (xb0, xb1, xb2, xb3)
        pebuf = (pb0, pb1)
        sem_x = (sx0, sx1, sx2, sx3)
        sem_pe = (sp0, sp1)
        sem_out = (so0, so1, so2, so3)

        wid = lax.axis_index("s") * _NUM_CORES + lax.axis_index("c")
        pe_base = wid * pe_rows_w * d

        def x_off(t):
            c, b = divmod(t, b_sc)
            return b * (s * d) + pe_base + c * blk

        descs = {}

        def issue_in_pe(c):
            descs[("pe", c)] = pltpu.async_copy(
                pe_hbm.at[pl.ds(pe_base + c * blk, blk)],
                pebuf[c % 2], sem_pe[c % 2])

        def issue_in_x(t):
            descs[("x", t)] = pltpu.async_copy(
                x_hbm.at[pl.ds(x_off(t), blk)], xbuf[t % 4], sem_x[t % 4])

        def issue_out(t):
            descs[("out", t)] = pltpu.async_copy(
                xbuf[t % 4], o_hbm.at[pl.ds(x_off(t), blk)], sem_out[t % 4])

        issue_in_pe(0)
        for t in range(min(3, T)):
            issue_in_x(t)

        for t in range(T):
            c, b = divmod(t, b_sc)
            if b == 0:
                if c + 1 < n_chunks:
                    issue_in_pe(c + 1)
                descs[("pe", c)].wait()
            descs[("x", t)].wait()

            xb = xbuf[t % 4]
            pb = pebuf[c % 2]

            @plsc.parallel_loop(0, blk // 16, unroll=8)
            def _(k):
                sl = pl.ds(k * 16, 16)
                plsc.addupdate(xb.at[sl], pb[sl])

            issue_out(t)
            if t + 3 < T:
                if t - 1 >= 0:
                    descs[("out", t - 1)].wait()
                issue_in_x(t + 3)
        for t in range(max(0, T - 4), T):
            descs[("out", t)].wait()

    mesh = plsc.VectorSubcoreMesh(
        core_axis_name="c",
        subcore_axis_name="s",
        num_cores=_NUM_CORES,
        num_subcores=_NUM_SUBCORES,
    )
    return pl.kernel(
        body,
        out_type=jax.ShapeDtypeStruct((b_sc * s * d,), jnp.float32),
        mesh=mesh,
        scratch_types=(
            [pltpu.VMEM((blk,), jnp.float32) for _ in range(4)]
            + [pltpu.VMEM((blk,), jnp.float32) for _ in range(2)]
            + [pltpu.SemaphoreType.DMA for _ in range(10)]
        ),
    )(x_flat, pe_flat)


def _tc_add_kernel(x_ref, pe_ref, o_ref):
    o_ref[...] = x_ref[...] + pe_ref[...]


def _tc_add(x, pe):
    B, S, D = x.shape
    S_BLK = 2048
    return pl.pallas_call(
        _tc_add_kernel,
        grid=(S // S_BLK, B),
        in_specs=[
            pl.BlockSpec((1, S_BLK, D), lambda i, j: (j, i, 0)),
            pl.BlockSpec((S_BLK, D), lambda i, j: (i, 0)),
        ],
        out_specs=pl.BlockSpec((1, S_BLK, D), lambda i, j: (j, i, 0)),
        out_shape=jax.ShapeDtypeStruct(x.shape, x.dtype),
        compiler_params=pltpu.CompilerParams(vmem_limit_bytes=60 * 1024 * 1024),
    )(x, pe)


_TC_R = 2048


def _tc_add_manual(x2d, pe, b, s, d):
    n_chunks = s // _TC_R
    T = n_chunks * b

    def body(x_hbm, pe_hbm, o_hbm, xb0, xb1, xb2, xb3, xb4, pb0, pb1,
             sx0, sx1, sx2, sx3, sx4, sp0, sp1, so0, so1, so2, so3, so4):
        xbuf = (xb0, xb1, xb2, xb3, xb4)
        pebuf = (pb0, pb1)
        sem_x = (sx0, sx1, sx2, sx3, sx4)
        sem_pe = (sp0, sp1)
        sem_out = (so0, so1, so2, so3, so4)

        def x_row(t):
            c, bb = divmod(t, b)
            return bb * s + c * _TC_R

        descs = {}

        def issue_in_pe(c):
            descs[("pe", c)] = pltpu.make_async_copy(
                pe_hbm.at[pl.ds(c * _TC_R, _TC_R)], pebuf[c % 2], sem_pe[c % 2])
            descs[("pe", c)].start()

        def issue_in_x(t):
            descs[("x", t)] = pltpu.make_async_copy(
                x_hbm.at[pl.ds(x_row(t), _TC_R)], xbuf[t % 5], sem_x[t % 5])
            descs[("x", t)].start()

        def issue_out(t):
            descs[("out", t)] = pltpu.make_async_copy(
                xbuf[t % 5], o_hbm.at[pl.ds(x_row(t), _TC_R)], sem_out[t % 5])
            descs[("out", t)].start()

        issue_in_pe(0)
        for t in range(min(4, T)):
            issue_in_x(t)

        for t in range(T):
            c, bb = divmod(t, b)
            if bb == 0:
                if c + 1 < n_chunks:
                    issue_in_pe(c + 1)
                descs[("pe", c)].wait()
            descs[("x", t)].wait()
            xbuf[t % 5][...] = xbuf[t % 5][...] + pebuf[c % 2][...]
            issue_out(t)
            if t + 4 < T:
                if t - 1 >= 0:
                    descs[("out", t - 1)].wait()
                issue_in_x(t + 4)
        for t in range(max(0, T - 5), T):
            descs[("out", t)].wait()

    return pl.pallas_call(
        body,
        in_specs=[
            pl.BlockSpec(memory_space=pl.ANY),
            pl.BlockSpec(memory_space=pl.ANY),
        ],
        out_specs=pl.BlockSpec(memory_space=pl.ANY),
        out_shape=jax.ShapeDtypeStruct((b * s, d), jnp.float32),
        scratch_shapes=(
            [pltpu.VMEM((_TC_R, d), jnp.float32) for _ in range(5)]
            + [pltpu.VMEM((_TC_R, d), jnp.float32) for _ in range(2)]
            + [pltpu.SemaphoreType.DMA for _ in range(12)]
        ),
        compiler_params=pltpu.CompilerParams(vmem_limit_bytes=60 * 1024 * 1024),
    )(x2d, pe)


def kernel(x, pe):
    B, S, D = x.shape
    out2d = _tc_add_manual(x.reshape(B * S, D), pe, B, S, D)
    return out2d.reshape(B, S, D)


# speedup vs baseline: 1.0026x; 1.0026x over previous
import jax
import jax.numpy as jnp
from jax.experimental import pallas as pl
from jax.experimental.pallas import tpu as pltpu

_TC_R = 2048


def _tc_add_manual(x2d, pe, b, s, d):
    n_chunks = s // _TC_R
    T = n_chunks * b

    def body(x_hbm, pe_hbm, o_hbm, xb0, xb1, xb2, xb3, xb4, pb0, pb1,
             sx0, sx1, sx2, sx3, sx4, sp0, sp1, so0, so1, so2, so3, so4):
        xbuf = (xb0, xb1, xb2, xb3, xb4)
        pebuf = (pb0, pb1)
        sem_x = (sx0, sx1, sx2, sx3, sx4)
        sem_pe = (sp0, sp1)
        sem_out = (so0, so1, so2, so3, so4)

        def x_row(t):
            c, bb = divmod(t, b)
            return bb * s + c * _TC_R

        descs = {}

        def issue_in_pe(c):
            descs[("pe", c)] = pltpu.make_async_copy(
                pe_hbm.at[pl.ds(c * _TC_R, _TC_R)], pebuf[c % 2], sem_pe[c % 2])
            descs[("pe", c)].start()

        def issue_in_x(t):
            descs[("x", t)] = pltpu.make_async_copy(
                x_hbm.at[pl.ds(x_row(t), _TC_R)], xbuf[t % 5], sem_x[t % 5])
            descs[("x", t)].start()

        def issue_out(t):
            descs[("out", t)] = pltpu.make_async_copy(
                xbuf[t % 5], o_hbm.at[pl.ds(x_row(t), _TC_R)], sem_out[t % 5])
            descs[("out", t)].start()

        issue_in_pe(0)
        for t in range(min(4, T)):
            issue_in_x(t)

        for t in range(T):
            c, bb = divmod(t, b)
            if bb == 0:
                if c + 1 < n_chunks:
                    issue_in_pe(c + 1)
                descs[("pe", c)].wait()
            descs[("x", t)].wait()
            xbuf[t % 5][...] = xbuf[t % 5][...] + pebuf[c % 2][...]
            issue_out(t)
            if t + 4 < T:
                if t - 1 >= 0:
                    descs[("out", t - 1)].wait()
                issue_in_x(t + 4)
        for t in range(max(0, T - 5), T):
            descs[("out", t)].wait()

    return pl.pallas_call(
        body,
        in_specs=[
            pl.BlockSpec(memory_space=pl.ANY),
            pl.BlockSpec(memory_space=pl.ANY),
        ],
        out_specs=pl.BlockSpec(memory_space=pl.ANY),
        out_shape=jax.ShapeDtypeStruct((b * s, d), jnp.float32),
        scratch_shapes=(
            [pltpu.VMEM((_TC_R, d), jnp.float32) for _ in range(5)]
            + [pltpu.VMEM((_TC_R, d), jnp.float32) for _ in range(2)]
            + [pltpu.SemaphoreType.DMA for _ in range(12)]
        ),
        compiler_params=pltpu.CompilerParams(vmem_limit_bytes=60 * 1024 * 1024),
    )(x2d, pe)


def kernel(x, pe):
    B, S, D = x.shape
    out2d = _tc_add_manual(x.reshape(B * S, D), pe, B, S, D)
    return out2d.reshape(B, S, D)
